# baseline (speedup 1.0000x reference)
import jax
import jax.numpy as jnp
from jax.experimental import pallas as pl
from jax.experimental.pallas import tpu as pltpu

_ROWS = 32768
_HID = 768
_EXP = 8
_BLOCK = 2048


def _gate_block(w_ref, thr_ref, x_ref, rw_ref, s_ref, k_ref):
    w = w_ref[...]
    wn = w / jnp.maximum(
        jnp.sqrt(jnp.sum(w * w, axis=0, keepdims=True)), 1e-12
    )
    x = x_ref[...]
    ss = jnp.sum(x * x, axis=1, keepdims=True)
    xn = x / jnp.maximum(jnp.sqrt(ss), 1e-12)
    scores = jax.lax.dot_general(
        xn.astype(jnp.bfloat16), wn.astype(jnp.bfloat16),
        (((1,), (0,)), ((), ())),
        preferred_element_type=jnp.float32,
    )
    thr = thr_ref[0, 0]
    mask = scores > thr
    k_ref[...] = jnp.sum(mask.astype(jnp.int32), axis=1, keepdims=True)
    masked = jnp.where(mask, scores, jnp.float32(-1e9))
    m = jnp.max(masked, axis=1, keepdims=True)
    e = jnp.exp(masked - m)
    rw_ref[...] = e / jnp.sum(e, axis=1, keepdims=True)
    s_ref[...] = scores


def kernel(hidden_states, sim_matrix, threshold):
    thr2 = threshold.reshape(1, 1)
    grid = (_ROWS // _BLOCK,)
    rw, s, k = pl.pallas_call(
        _gate_block,
        grid=grid,
        in_specs=[
            pl.BlockSpec((_HID, _EXP), lambda i: (0, 0)),
            pl.BlockSpec((1, 1), lambda i: (0, 0)),
            pl.BlockSpec((_BLOCK, _HID), lambda i: (i, 0)),
        ],
        out_specs=[
            pl.BlockSpec((_BLOCK, _EXP), lambda i: (i, 0)),
            pl.BlockSpec((_BLOCK, _EXP), lambda i: (i, 0)),
            pl.BlockSpec((_BLOCK, 1), lambda i: (i, 0)),
        ],
        out_shape=[
            jax.ShapeDtypeStruct((_ROWS, _EXP), jnp.float32),
            jax.ShapeDtypeStruct((_ROWS, _EXP), jnp.float32),
            jax.ShapeDtypeStruct((_ROWS, 1), jnp.int32),
        ],
        compiler_params=pltpu.CompilerParams(
            dimension_semantics=("arbitrary",),
        ),
    )(sim_matrix, thr2, hidden_states)
    return rw, s, k.reshape(_ROWS)


# speedup vs baseline: 1.0374x; 1.0374x over previous
import jax
import jax.numpy as jnp
from jax.experimental import pallas as pl
from jax.experimental.pallas import tpu as pltpu

_ROWS = 32768
_HID = 768
_EXP = 8
_BLOCK = 4096


def _gate_block(w_ref, thr_ref, x_ref, rw_ref, s_ref, k_ref):
    w = w_ref[...]
    wn = w / jnp.maximum(
        jnp.sqrt(jnp.sum(w * w, axis=0, keepdims=True)), 1e-12
    )
    x = x_ref[...]
    ss = jnp.sum(x * x, axis=1, keepdims=True)
    xn = x / jnp.maximum(jnp.sqrt(ss), 1e-12)
    scores = jax.lax.dot_general(
        xn.astype(jnp.bfloat16), wn.astype(jnp.bfloat16),
        (((1,), (0,)), ((), ())),
        preferred_element_type=jnp.float32,
    )
    thr = thr_ref[0, 0]
    mask = scores > thr
    k_ref[...] = jnp.sum(mask.astype(jnp.int32), axis=1, keepdims=True)
    masked = jnp.where(mask, scores, jnp.float32(-1e9))
    m = jnp.max(masked, axis=1, keepdims=True)
    e = jnp.exp(masked - m)
    rw_ref[...] = e / jnp.sum(e, axis=1, keepdims=True)
    s_ref[...] = scores


def kernel(hidden_states, sim_matrix, threshold):
    thr2 = threshold.reshape(1, 1)
    grid = (_ROWS // _BLOCK,)
    rw, s, k = pl.pallas_call(
        _gate_block,
        grid=grid,
        in_specs=[
            pl.BlockSpec((_HID, _EXP), lambda i: (0, 0)),
            pl.BlockSpec((1, 1), lambda i: (0, 0)),
            pl.BlockSpec((_BLOCK, _HID), lambda i: (i, 0)),
        ],
        out_specs=[
            pl.BlockSpec((_BLOCK, _EXP), lambda i: (i, 0)),
            pl.BlockSpec((_BLOCK, _EXP), lambda i: (i, 0)),
            pl.BlockSpec((_BLOCK, 1), lambda i: (i, 0)),
        ],
        out_shape=[
            jax.ShapeDtypeStruct((_ROWS, _EXP), jnp.float32),
            jax.ShapeDtypeStruct((_ROWS, _EXP), jnp.float32),
            jax.ShapeDtypeStruct((_ROWS, 1), jnp.int32),
        ],
        compiler_params=pltpu.CompilerParams(
            dimension_semantics=("arbitrary",),
        ),
    )(sim_matrix, thr2, hidden_states)
    return rw, s, k.reshape(_ROWS)
